# baseline (speedup 1.0000x reference)
import functools

import jax
import jax.numpy as jnp
from jax import lax
from jax.experimental import pallas as pl
from jax.experimental.pallas import tpu as pltpu
from jax.experimental.pallas import tpu_sc as plsc

N = 10000
D = 256
DH = 128
E = 160000

NC = 2
NS = 16
CH = 64

N_PAD = 10240
ROWS_PER_TILE = N_PAD // NS
DROWS = N_PAD // 128
E_PAD = 163840
CHUNKS = E_PAD // (NS * CH)
EROWS = E_PAD // CH
SUP = 16
NSUP = CHUNKS // SUP


def _sc_aggregate(xflat, srcm, dstm, zeros80, iota8):
    mesh = plsc.VectorSubcoreMesh(core_axis_name="c", subcore_axis_name="s")

    @functools.partial(
        pl.kernel,
        out_type=(
            jax.ShapeDtypeStruct((N_PAD, D), jnp.float32),
            jax.ShapeDtypeStruct((NC, DROWS, 128), jnp.float32),
        ),
        mesh=mesh,
        compiler_params=pltpu.CompilerParams(needs_layout_passes=False),
        scratch_types=[
            pltpu.VMEM_SHARED((N_PAD, DH), jnp.float32),
            pltpu.VMEM_SHARED((DROWS, 128), jnp.float32),
            pltpu.VMEM((SUP, CH), jnp.int32),
            pltpu.VMEM((SUP, CH), jnp.int32),
            pltpu.VMEM((CH, DH), jnp.float32),
            pltpu.VMEM((CH, DH), jnp.float32),
            pltpu.VMEM((DROWS, 128), jnp.float32),
            pltpu.VMEM((8, DROWS), jnp.int32),
            pltpu.SemaphoreType.DMA,
            pltpu.SemaphoreType.DMA,
            pltpu.SemaphoreType.DMA,
            pltpu.SemaphoreType.DMA,
        ],
    )
    def sc_kernel(xflat_h, srcm_h, dstm_h, zeros80_h, iota8_h,
                  agg_h, deg_h, acc, degacc, srcv, dstv, rows0, rows1,
                  degl, iotav, sem0, sem1, sem2, sem3):
        c = lax.axis_index("c")
        s = lax.axis_index("s")
        rbase = s * ROWS_PER_TILE
        bufs = (rows0, rows1)
        gsems = (sem0, sem1)
        ssems = (sem2, sem3)
        pltpu.sync_copy(zeros80_h, degl)
        pltpu.sync_copy(iota8_h, iotav)

        @pl.loop(0, ROWS_PER_TILE // DROWS)
        def zbody(i):
            off = pl.multiple_of(rbase + i * DROWS, 8)
            pltpu.sync_copy(degl, acc.at[pl.ds(off, DROWS)])

        @pl.when(s < DROWS // 8)
        def _():
            zoff = pl.multiple_of(s * 8, 8)
            pltpu.sync_copy(degl.at[pl.ds(0, 8)], degacc.at[pl.ds(zoff, 8)])

        plsc.subcore_barrier()

        sbase = c * EROWS + s * CHUNKS
        dbase = s * CHUNKS
        one16 = jnp.full((16,), 1.0, jnp.float32)

        @pl.loop(0, NSUP)
        def body(j):
            soff = pl.multiple_of(sbase + j * SUP, 8)
            doff = pl.multiple_of(dbase + j * SUP, 8)
            pltpu.sync_copy(srcm_h.at[pl.ds(soff, SUP)], srcv)
            pltpu.sync_copy(dstm_h.at[pl.ds(doff, SUP)], dstv)
            gd = [None, None]
            sd = [None, None]
            gd[0] = pltpu.async_copy(xflat_h.at[srcv.at[0]], bufs[0],
                                     gsems[0])
            for k in range(SUP):
                b = k % 2
                gd[b].wait()
                if k > 0:
                    sd[1 - b].wait()
                sd[b] = pltpu.async_copy(bufs[b], acc.at[dstv.at[k]],
                                         ssems[b], add=True)
                if k < SUP - 1:
                    gd[1 - b] = pltpu.async_copy(
                        xflat_h.at[srcv.at[k + 1]], bufs[1 - b],
                        gsems[1 - b])

                @pl.when(c == (k % 2))
                def _():
                    for t in range(CH // 16):
                        v = dstv[k, pl.ds(t * 16, 16)]
                        hi = lax.shift_right_logical(v, 7)
                        lo = lax.bitwise_and(v, 127)
                        plsc.addupdate_scatter(degl, [hi, lo], one16)
            sd[(SUP - 1) % 2].wait()

        pltpu.sync_copy(degl, degacc.at[iotav.at[0]], add=True)
        plsc.subcore_barrier()

        coff = pl.multiple_of(c * DH, 8)
        npieces = ROWS_PER_TILE // CH
        rd = [None, None]
        wr = [None, None]
        rd[0] = pltpu.async_copy(acc.at[pl.ds(rbase, CH)], bufs[0], gsems[0])
        for i in range(npieces):
            b = i % 2
            off = pl.multiple_of(rbase + i * CH, 8)
            rd[b].wait()
            wr[b] = pltpu.async_copy(
                bufs[b], agg_h.at[pl.ds(off, CH), pl.ds(coff, DH)], ssems[b])
            if i < npieces - 1:
                if i > 0:
                    wr[1 - b].wait()
                noff = pl.multiple_of(rbase + (i + 1) * CH, 8)
                rd[1 - b] = pltpu.async_copy(acc.at[pl.ds(noff, CH)],
                                             bufs[1 - b], gsems[1 - b])
        wr[0].wait()
        wr[1].wait()

        @pl.when(s < DROWS // 8)
        def _():
            zoff = pl.multiple_of(s * 8, 8)
            pltpu.sync_copy(degacc.at[pl.ds(zoff, 8)], rows1.at[pl.ds(0, 8)])
            pltpu.sync_copy(rows1.at[pl.ds(0, 8)], deg_h.at[c, pl.ds(zoff, 8)])

    return sc_kernel(xflat, srcm, dstm, zeros80, iota8)


def _tc_finish(agg, degflat, W, b2):
    def body(agg_ref, deg_ref, w_ref, b_ref, out_ref):
        deg = deg_ref[0] + deg_ref[1]
        dinv = jnp.where(deg > 0, 1.0 / deg, 0.0)
        a = agg_ref[...] * dinv[:, None]
        out_ref[...] = jnp.dot(a, w_ref[...],
                               preferred_element_type=jnp.float32) + b_ref[...]

    blk = 2048
    return pl.pallas_call(
        body,
        grid=(N_PAD // blk,),
        in_specs=[
            pl.BlockSpec((blk, D), lambda i: (i, 0)),
            pl.BlockSpec((NC, blk), lambda i: (0, i)),
            pl.BlockSpec((D, D), lambda i: (0, 0)),
            pl.BlockSpec((1, D), lambda i: (0, 0)),
        ],
        out_specs=pl.BlockSpec((blk, D), lambda i: (i, 0)),
        out_shape=jax.ShapeDtypeStruct((N_PAD, D), jnp.float32),
    )(agg, degflat, W, b2)


def kernel(x, edge_index, W, b):
    src = edge_index[0]
    dst = edge_index[1]
    pad = E_PAD - E
    src_p = jnp.concatenate([src, jnp.zeros((pad,), jnp.int32)])
    dst_p = jnp.concatenate(
        [dst, N + (jnp.arange(pad, dtype=jnp.int32) % (N_PAD - N))])
    srcm = jnp.concatenate([src_p, src_p + N]).reshape(2 * EROWS, CH)
    dstm = dst_p.reshape(EROWS, CH)
    xflat = jnp.concatenate([x[:, :DH], x[:, DH:]], axis=0)
    zeros80 = jnp.zeros((DROWS, 128), jnp.float32)
    iota8 = jnp.broadcast_to(jnp.arange(DROWS, dtype=jnp.int32), (8, DROWS))

    agg, degb = _sc_aggregate(xflat, srcm, dstm, zeros80, iota8)
    out = _tc_finish(agg, degb.reshape(NC, N_PAD), W, b.reshape(1, D))
    return out[:N]


# speedup vs baseline: 1.0489x; 1.0489x over previous
import functools

import jax
import jax.numpy as jnp
from jax import lax
from jax.experimental import pallas as pl
from jax.experimental.pallas import tpu as pltpu
from jax.experimental.pallas import tpu_sc as plsc

N = 10000
D = 256
DH = 128
E = 160000

NC = 2
NS = 16
CH = 80

N_PAD = 10240
ROWS_PER_TILE = N_PAD // NS
DROWS = N_PAD // 128
E_PAD = 163840
CHUNKS = E_PAD // (NS * CH)
EROWS = E_PAD // CH
SUP = 16
NSUP = CHUNKS // SUP


def _sc_aggregate(xflat, srcm, dstm, zeros80, iota8):
    mesh = plsc.VectorSubcoreMesh(core_axis_name="c", subcore_axis_name="s")

    @functools.partial(
        pl.kernel,
        out_type=(
            jax.ShapeDtypeStruct((N_PAD, D), jnp.float32),
            jax.ShapeDtypeStruct((NC, DROWS, 128), jnp.float32),
        ),
        mesh=mesh,
        compiler_params=pltpu.CompilerParams(needs_layout_passes=False),
        scratch_types=[
            pltpu.VMEM_SHARED((N_PAD, DH), jnp.float32),
            pltpu.VMEM_SHARED((DROWS, 128), jnp.float32),
            pltpu.VMEM((SUP, CH), jnp.int32),
            pltpu.VMEM((SUP, CH), jnp.int32),
            pltpu.VMEM((CH, DH), jnp.float32),
            pltpu.VMEM((CH, DH), jnp.float32),
            pltpu.VMEM((DROWS, 128), jnp.float32),
            pltpu.VMEM((8, DROWS), jnp.int32),
            pltpu.SemaphoreType.DMA,
            pltpu.SemaphoreType.DMA,
            pltpu.SemaphoreType.DMA,
            pltpu.SemaphoreType.DMA,
        ],
    )
    def sc_kernel(xflat_h, srcm_h, dstm_h, zeros80_h, iota8_h,
                  agg_h, deg_h, acc, degacc, srcv, dstv, rows0, rows1,
                  degl, iotav, sem0, sem1, sem2, sem3):
        c = lax.axis_index("c")
        s = lax.axis_index("s")
        rbase = s * ROWS_PER_TILE
        bufs = (rows0, rows1)
        gsems = (sem0, sem1)
        ssems = (sem2, sem3)
        pltpu.sync_copy(zeros80_h, degl)
        pltpu.sync_copy(iota8_h, iotav)

        @pl.loop(0, ROWS_PER_TILE // DROWS)
        def zbody(i):
            off = pl.multiple_of(rbase + i * DROWS, 8)
            pltpu.sync_copy(degl, acc.at[pl.ds(off, DROWS)])

        @pl.when(s < DROWS // 8)
        def _():
            zoff = pl.multiple_of(s * 8, 8)
            pltpu.sync_copy(degl.at[pl.ds(0, 8)], degacc.at[pl.ds(zoff, 8)])

        plsc.subcore_barrier()

        sbase = c * EROWS + s * CHUNKS
        dbase = s * CHUNKS
        one16 = jnp.full((16,), 1.0, jnp.float32)

        @pl.loop(0, NSUP)
        def body(j):
            soff = pl.multiple_of(sbase + j * SUP, 8)
            doff = pl.multiple_of(dbase + j * SUP, 8)
            pltpu.sync_copy(srcm_h.at[pl.ds(soff, SUP)], srcv)
            pltpu.sync_copy(dstm_h.at[pl.ds(doff, SUP)], dstv)
            gd = [None, None]
            sd = [None, None]
            gd[0] = pltpu.async_copy(xflat_h.at[srcv.at[0]], bufs[0],
                                     gsems[0])
            for k in range(SUP):
                b = k % 2
                gd[b].wait()
                if k > 0:
                    sd[1 - b].wait()
                sd[b] = pltpu.async_copy(bufs[b], acc.at[dstv.at[k]],
                                         ssems[b], add=True)
                if k < SUP - 1:
                    gd[1 - b] = pltpu.async_copy(
                        xflat_h.at[srcv.at[k + 1]], bufs[1 - b],
                        gsems[1 - b])

                @pl.when(c == (k % 2))
                def _():
                    for t in range(CH // 16):
                        v = dstv[k, pl.ds(t * 16, 16)]
                        hi = lax.shift_right_logical(v, 7)
                        lo = lax.bitwise_and(v, 127)
                        plsc.addupdate_scatter(degl, [hi, lo], one16)
            sd[(SUP - 1) % 2].wait()

        pltpu.sync_copy(degl, degacc.at[iotav.at[0]], add=True)
        plsc.subcore_barrier()

        coff = pl.multiple_of(c * DH, 8)
        npieces = ROWS_PER_TILE // CH
        rd = [None, None]
        wr = [None, None]
        rd[0] = pltpu.async_copy(acc.at[pl.ds(rbase, CH)], bufs[0], gsems[0])
        for i in range(npieces):
            b = i % 2
            off = pl.multiple_of(rbase + i * CH, 8)
            rd[b].wait()
            wr[b] = pltpu.async_copy(
                bufs[b], agg_h.at[pl.ds(off, CH), pl.ds(coff, DH)], ssems[b])
            if i < npieces - 1:
                if i > 0:
                    wr[1 - b].wait()
                noff = pl.multiple_of(rbase + (i + 1) * CH, 8)
                rd[1 - b] = pltpu.async_copy(acc.at[pl.ds(noff, CH)],
                                             bufs[1 - b], gsems[1 - b])
        wr[0].wait()
        wr[1].wait()

        @pl.when(s < DROWS // 8)
        def _():
            zoff = pl.multiple_of(s * 8, 8)
            pltpu.sync_copy(degacc.at[pl.ds(zoff, 8)], rows1.at[pl.ds(0, 8)])
            pltpu.sync_copy(rows1.at[pl.ds(0, 8)], deg_h.at[c, pl.ds(zoff, 8)])

    return sc_kernel(xflat, srcm, dstm, zeros80, iota8)


def _tc_finish(agg, degflat, W, b2):
    def body(agg_ref, deg_ref, w_ref, b_ref, out_ref):
        deg = deg_ref[0] + deg_ref[1]
        dinv = jnp.where(deg > 0, 1.0 / deg, 0.0)
        a = agg_ref[...] * dinv[:, None]
        out_ref[...] = jnp.dot(a, w_ref[...],
                               preferred_element_type=jnp.float32) + b_ref[...]

    blk = 2048
    return pl.pallas_call(
        body,
        grid=(N_PAD // blk,),
        in_specs=[
            pl.BlockSpec((blk, D), lambda i: (i, 0)),
            pl.BlockSpec((NC, blk), lambda i: (0, i)),
            pl.BlockSpec((D, D), lambda i: (0, 0)),
            pl.BlockSpec((1, D), lambda i: (0, 0)),
        ],
        out_specs=pl.BlockSpec((blk, D), lambda i: (i, 0)),
        out_shape=jax.ShapeDtypeStruct((N_PAD, D), jnp.float32),
    )(agg, degflat, W, b2)


def kernel(x, edge_index, W, b):
    src = edge_index[0]
    dst = edge_index[1]
    pad = E_PAD - E
    src_p = jnp.concatenate([src, jnp.zeros((pad,), jnp.int32)])
    dst_p = jnp.concatenate(
        [dst, N + (jnp.arange(pad, dtype=jnp.int32) % (N_PAD - N))])
    srcm = jnp.concatenate([src_p, src_p + N]).reshape(2 * EROWS, CH)
    dstm = dst_p.reshape(EROWS, CH)
    xflat = jnp.concatenate([x[:, :DH], x[:, DH:]], axis=0)
    zeros80 = jnp.zeros((DROWS, 128), jnp.float32)
    iota8 = jnp.broadcast_to(jnp.arange(DROWS, dtype=jnp.int32), (8, DROWS))

    agg, degb = _sc_aggregate(xflat, srcm, dstm, zeros80, iota8)
    out = _tc_finish(agg, degb.reshape(NC, N_PAD), W, b.reshape(1, D))
    return out[:N]
